# baseline (device time: 114481 ns/iter reference)
import jax
import jax.numpy as jnp
from jax import lax
from jax.experimental import pallas as pl
from jax.experimental.pallas import tpu as pltpu

N_DEV = 32
N_ROUNDS = 5
B, SQ, D, HQ, DH = 2, 128, 512, 8, 64
SKV = 128
BH = B * HQ
SCALE = 0.125


def kernel(x, Wq, Wo, K_ext, V_ext):
    def body(
        x_ref, wq_ref, wo_ref, k_ref, v_ref,
        out_ref,
        acc_o, acc_ml, recv_o, recv_ml, attn_ref,
        send_o_sems, recv_o_sems, send_ml_sems, recv_ml_sems,
    ):
        my = lax.axis_index("i")

        xm = x_ref[:].reshape(B * SQ, D)
        q = jnp.dot(xm, wq_ref[:], preferred_element_type=jnp.float32)
        for b in range(B):
            for h in range(HQ):
                idx = b * HQ + h
                q_bh = q[b * SQ:(b + 1) * SQ, h * DH:(h + 1) * DH]
                k_bh = k_ref[b, :, h, :]
                v_bh = v_ref[b, :, h, :]
                s = lax.dot_general(
                    q_bh, k_bh, (((1,), (1,)), ((), ())),
                    preferred_element_type=jnp.float32,
                ) * SCALE
                m = jnp.max(s, axis=1)
                p = jnp.exp(s - m[:, None])
                l = jnp.sum(p, axis=1)
                o = jnp.dot(p, v_bh, preferred_element_type=jnp.float32)
                acc_o[idx] = o
                acc_ml[0, idx] = m
                acc_ml[1, idx] = l

        for r in range(N_ROUNDS):
            partner = my ^ (1 << r)
            rdma_o = pltpu.make_async_remote_copy(
                src_ref=acc_o,
                dst_ref=recv_o.at[r],
                send_sem=send_o_sems.at[r],
                recv_sem=recv_o_sems.at[r],
                device_id=(partner,),
                device_id_type=pl.DeviceIdType.MESH,
            )
            rdma_ml = pltpu.make_async_remote_copy(
                src_ref=acc_ml,
                dst_ref=recv_ml.at[r],
                send_sem=send_ml_sems.at[r],
                recv_sem=recv_ml_sems.at[r],
                device_id=(partner,),
                device_id_type=pl.DeviceIdType.MESH,
            )
            rdma_o.start()
            rdma_ml.start()
            rdma_o.wait()
            rdma_ml.wait()

            m_a = acc_ml[0]
            l_a = acc_ml[1]
            m_b = recv_ml[r, 0]
            l_b = recv_ml[r, 1]
            m_n = jnp.maximum(m_a, m_b)
            alpha = jnp.exp(m_a - m_n)
            beta = jnp.exp(m_b - m_n)
            acc_ml[0] = m_n
            acc_ml[1] = l_a * alpha + l_b * beta
            acc_o[:] = (
                acc_o[:] * alpha[:, :, None] + recv_o[r] * beta[:, :, None]
            )

        o_norm = acc_o[:] / acc_ml[1][:, :, None]
        for b in range(B):
            for h in range(HQ):
                idx = b * HQ + h
                attn_ref[b * SQ:(b + 1) * SQ, h * DH:(h + 1) * DH] = o_norm[idx]
        out = jnp.dot(attn_ref[:], wo_ref[:], preferred_element_type=jnp.float32)
        out_ref[:] = out.reshape(B, SQ, D)

    return pl.pallas_call(
        body,
        out_shape=jax.ShapeDtypeStruct((B, SQ, D), jnp.float32),
        in_specs=[pl.BlockSpec(memory_space=pltpu.VMEM)] * 5,
        out_specs=pl.BlockSpec(memory_space=pltpu.VMEM),
        scratch_shapes=[
            pltpu.VMEM((BH, SQ, DH), jnp.float32),
            pltpu.VMEM((2, BH, SQ), jnp.float32),
            pltpu.VMEM((N_ROUNDS, BH, SQ, DH), jnp.float32),
            pltpu.VMEM((N_ROUNDS, 2, BH, SQ), jnp.float32),
            pltpu.VMEM((B * SQ, HQ * DH), jnp.float32),
            pltpu.SemaphoreType.DMA((N_ROUNDS,)),
            pltpu.SemaphoreType.DMA((N_ROUNDS,)),
            pltpu.SemaphoreType.DMA((N_ROUNDS,)),
            pltpu.SemaphoreType.DMA((N_ROUNDS,)),
        ],
    )(x, Wq, Wo, K_ext, V_ext)


# device time: 67828 ns/iter; 1.6878x vs baseline; 1.6878x over previous
import jax
import jax.numpy as jnp
from jax import lax
from jax.experimental import pallas as pl
from jax.experimental.pallas import tpu as pltpu

N_DEV = 32
N_ROUNDS = 5
B, SQ, D, HQ, DH = 2, 128, 512, 8, 64
SKV = 128
BH = B * HQ
SCALE = 0.125


def kernel(x, Wq, Wo, K_ext, V_ext):
    def body(
        x_ref, wq_ref, wo_ref, k_ref, v_ref,
        out_ref,
        acc_o, acc_ml, send_o, recv_o, recv_ml, attn_ref,
        send_o_sems, recv_o_sems, send_ml_sems, recv_ml_sems,
    ):
        my = lax.axis_index("i")

        barrier_sem = pltpu.get_barrier_semaphore()
        for r in range(N_ROUNDS):
            pl.semaphore_signal(
                barrier_sem, inc=1,
                device_id=(my ^ (1 << r),),
                device_id_type=pl.DeviceIdType.MESH,
            )
        pl.semaphore_wait(barrier_sem, N_ROUNDS)

        xm = x_ref[:].reshape(B * SQ, D)
        q = jnp.dot(xm, wq_ref[:], preferred_element_type=jnp.float32)
        for b in range(B):
            for h in range(HQ):
                idx = b * HQ + h
                q_bh = q[b * SQ:(b + 1) * SQ, h * DH:(h + 1) * DH]
                k_bh = k_ref[b, :, h, :]
                v_bh = v_ref[b, :, h, :]
                s = lax.dot_general(
                    q_bh, k_bh, (((1,), (1,)), ((), ())),
                    preferred_element_type=jnp.float32,
                ) * SCALE
                m = jnp.max(s, axis=1)
                p = jnp.exp(s - m[:, None])
                l = jnp.sum(p, axis=1)
                o = jnp.dot(p, v_bh, preferred_element_type=jnp.float32)
                acc_o[idx] = o
                send_o[idx] = o.astype(jnp.bfloat16)
                acc_ml[0, idx] = m
                acc_ml[1, idx] = l

        for r in range(N_ROUNDS):
            partner = my ^ (1 << r)
            rdma_o = pltpu.make_async_remote_copy(
                src_ref=send_o,
                dst_ref=recv_o.at[r],
                send_sem=send_o_sems.at[r],
                recv_sem=recv_o_sems.at[r],
                device_id=(partner,),
                device_id_type=pl.DeviceIdType.MESH,
            )
            rdma_ml = pltpu.make_async_remote_copy(
                src_ref=acc_ml,
                dst_ref=recv_ml.at[r],
                send_sem=send_ml_sems.at[r],
                recv_sem=recv_ml_sems.at[r],
                device_id=(partner,),
                device_id_type=pl.DeviceIdType.MESH,
            )
            rdma_o.start()
            rdma_ml.start()

            rdma_ml.wait()
            m_a = acc_ml[0]
            l_a = acc_ml[1]
            m_b = recv_ml[r, 0]
            l_b = recv_ml[r, 1]
            m_n = jnp.maximum(m_a, m_b)
            alpha = jnp.exp(m_a - m_n)
            beta = jnp.exp(m_b - m_n)
            acc_ml[0] = m_n
            acc_ml[1] = l_a * alpha + l_b * beta

            rdma_o.wait()
            merged = (
                acc_o[:] * alpha[:, :, None]
                + recv_o[r].astype(jnp.float32) * beta[:, :, None]
            )
            acc_o[:] = merged
            if r < N_ROUNDS - 1:
                send_o[:] = merged.astype(jnp.bfloat16)

        o_norm = acc_o[:] / acc_ml[1][:, :, None]
        for b in range(B):
            for h in range(HQ):
                idx = b * HQ + h
                attn_ref[b * SQ:(b + 1) * SQ, h * DH:(h + 1) * DH] = o_norm[idx]
        out = jnp.dot(attn_ref[:], wo_ref[:], preferred_element_type=jnp.float32)
        out_ref[:] = out.reshape(B, SQ, D)

    return pl.pallas_call(
        body,
        out_shape=jax.ShapeDtypeStruct((B, SQ, D), jnp.float32),
        in_specs=[pl.BlockSpec(memory_space=pltpu.VMEM)] * 5,
        out_specs=pl.BlockSpec(memory_space=pltpu.VMEM),
        scratch_shapes=[
            pltpu.VMEM((BH, SQ, DH), jnp.float32),
            pltpu.VMEM((2, BH, SQ), jnp.float32),
            pltpu.VMEM((BH, SQ, DH), jnp.bfloat16),
            pltpu.VMEM((N_ROUNDS, BH, SQ, DH), jnp.bfloat16),
            pltpu.VMEM((N_ROUNDS, 2, BH, SQ), jnp.float32),
            pltpu.VMEM((B * SQ, HQ * DH), jnp.float32),
            pltpu.SemaphoreType.DMA((N_ROUNDS,)),
            pltpu.SemaphoreType.DMA((N_ROUNDS,)),
            pltpu.SemaphoreType.DMA((N_ROUNDS,)),
            pltpu.SemaphoreType.DMA((N_ROUNDS,)),
        ],
        compiler_params=pltpu.CompilerParams(collective_id=0),
    )(x, Wq, Wo, K_ext, V_ext)


# device time: 54458 ns/iter; 2.1022x vs baseline; 1.2455x over previous
import jax
import jax.numpy as jnp
from jax import lax
from jax.experimental import pallas as pl
from jax.experimental.pallas import tpu as pltpu

N_DEV = 32
N_ROUNDS = 5
B, SQ, D, HQ, DH = 2, 128, 512, 8, 64
SKV = 128
BH = B * HQ
SCALE = 0.125


def kernel(x, Wq, Wo, K_ext, V_ext):
    def body(
        x_ref, wq_ref, wo_ref, k_ref, v_ref,
        out_ref,
        acc_o, acc_ml, send_o, recv_o, recv_ml, attn_ref,
        send_o_sems, recv_o_sems, send_ml_sems, recv_ml_sems,
    ):
        my = lax.axis_index("i")

        barrier_sem = pltpu.get_barrier_semaphore()
        for r in range(N_ROUNDS):
            pl.semaphore_signal(
                barrier_sem, inc=1,
                device_id=(my ^ (1 << r),),
                device_id_type=pl.DeviceIdType.MESH,
            )
        pl.semaphore_wait(barrier_sem, N_ROUNDS)

        xm = x_ref[:].reshape(B * SQ, D)
        q = jnp.dot(xm, wq_ref[:], preferred_element_type=jnp.float32)
        for b in range(B):
            for h in range(HQ):
                idx = b * HQ + h
                q_bh = q[b * SQ:(b + 1) * SQ, h * DH:(h + 1) * DH]
                k_bh = k_ref[b, :, h, :]
                v_bh = v_ref[b, :, h, :]
                s = lax.dot_general(
                    q_bh, k_bh, (((1,), (1,)), ((), ())),
                    preferred_element_type=jnp.float32,
                ) * SCALE
                m = jnp.max(s, axis=1)
                p = jnp.exp(s - m[:, None])
                l = jnp.sum(p, axis=1)
                o = jnp.dot(p, v_bh, preferred_element_type=jnp.float32)
                acc_o[idx] = o
                send_o[idx] = o.astype(jnp.bfloat16)
                acc_ml[0, idx] = m
                acc_ml[1, idx] = l

        HB = BH // 2

        def make_ml(r, partner):
            return pltpu.make_async_remote_copy(
                src_ref=acc_ml,
                dst_ref=recv_ml.at[r],
                send_sem=send_ml_sems.at[r],
                recv_sem=recv_ml_sems.at[r],
                device_id=(partner,),
                device_id_type=pl.DeviceIdType.MESH,
            )

        def make_o(r, c, partner):
            return pltpu.make_async_remote_copy(
                src_ref=send_o.at[pl.ds(c * HB, HB)],
                dst_ref=recv_o.at[r, pl.ds(c * HB, HB)],
                send_sem=send_o_sems.at[r, c],
                recv_sem=recv_o_sems.at[r, c],
                device_id=(partner,),
                device_id_type=pl.DeviceIdType.MESH,
            )

        partners = [my ^ (1 << r) for r in range(N_ROUNDS)]
        rdma_ml = {0: make_ml(0, partners[0])}
        rdma_o = {(0, 0): make_o(0, 0, partners[0]),
                  (0, 1): make_o(0, 1, partners[0])}
        rdma_ml[0].start()
        rdma_o[0, 0].start()
        rdma_o[0, 1].start()

        for r in range(N_ROUNDS):
            rdma_ml[r].wait()
            m_a = acc_ml[0]
            l_a = acc_ml[1]
            m_b = recv_ml[r, 0]
            l_b = recv_ml[r, 1]
            m_n = jnp.maximum(m_a, m_b)
            alpha = jnp.exp(m_a - m_n)
            beta = jnp.exp(m_b - m_n)
            acc_ml[0] = m_n
            acc_ml[1] = l_a * alpha + l_b * beta
            if r + 1 < N_ROUNDS:
                rdma_ml[r + 1] = make_ml(r + 1, partners[r + 1])
                rdma_ml[r + 1].start()

            for c in range(2):
                sl = pl.ds(c * HB, HB)
                rdma_o[r, c].wait()
                merged = (
                    acc_o[sl] * alpha[c * HB:(c + 1) * HB, :, None]
                    + recv_o[r, sl].astype(jnp.float32)
                    * beta[c * HB:(c + 1) * HB, :, None]
                )
                acc_o[sl] = merged
                if r + 1 < N_ROUNDS:
                    send_o[sl] = merged.astype(jnp.bfloat16)
                    rdma_o[r + 1, c] = make_o(r + 1, c, partners[r + 1])
                    rdma_o[r + 1, c].start()

        o_norm = acc_o[:] / acc_ml[1][:, :, None]
        for b in range(B):
            for h in range(HQ):
                idx = b * HQ + h
                attn_ref[b * SQ:(b + 1) * SQ, h * DH:(h + 1) * DH] = o_norm[idx]
        out = jnp.dot(attn_ref[:], wo_ref[:], preferred_element_type=jnp.float32)
        out_ref[:] = out.reshape(B, SQ, D)

    return pl.pallas_call(
        body,
        out_shape=jax.ShapeDtypeStruct((B, SQ, D), jnp.float32),
        in_specs=[pl.BlockSpec(memory_space=pltpu.VMEM)] * 5,
        out_specs=pl.BlockSpec(memory_space=pltpu.VMEM),
        scratch_shapes=[
            pltpu.VMEM((BH, SQ, DH), jnp.float32),
            pltpu.VMEM((2, BH, SQ), jnp.float32),
            pltpu.VMEM((BH, SQ, DH), jnp.bfloat16),
            pltpu.VMEM((N_ROUNDS, BH, SQ, DH), jnp.bfloat16),
            pltpu.VMEM((N_ROUNDS, 2, BH, SQ), jnp.float32),
            pltpu.VMEM((B * SQ, HQ * DH), jnp.float32),
            pltpu.SemaphoreType.DMA((N_ROUNDS, 2)),
            pltpu.SemaphoreType.DMA((N_ROUNDS, 2)),
            pltpu.SemaphoreType.DMA((N_ROUNDS,)),
            pltpu.SemaphoreType.DMA((N_ROUNDS,)),
        ],
        compiler_params=pltpu.CompilerParams(collective_id=0),
    )(x, Wq, Wo, K_ext, V_ext)


# device time: 51222 ns/iter; 2.2350x vs baseline; 1.0632x over previous
import jax
import jax.numpy as jnp
from jax import lax
from jax.experimental import pallas as pl
from jax.experimental.pallas import tpu as pltpu

N_DEV = 32
N_ROUNDS = 5
B, SQ, D, HQ, DH = 2, 128, 512, 8, 64
SKV = 128
BH = B * HQ
SCALE = 0.125


def kernel(x, Wq, Wo, K_ext, V_ext):
    def body(
        x_ref, wq_ref, wo_ref, k_ref, v_ref,
        out_ref,
        acc_o, acc_ml, send_o, recv_o, recv_ml, attn_ref,
        send_o_sems, recv_o_sems, send_ml_sems, recv_ml_sems,
    ):
        my = lax.axis_index("i")

        barrier_sem = pltpu.get_barrier_semaphore()
        for r in range(N_ROUNDS):
            pl.semaphore_signal(
                barrier_sem, inc=1,
                device_id=(my ^ (1 << r),),
                device_id_type=pl.DeviceIdType.MESH,
            )

        xm = x_ref[:].reshape(B * SQ, D)
        q = jnp.dot(xm, wq_ref[:], preferred_element_type=jnp.float32)

        def do_head(b, h):
            idx = b * HQ + h
            q_bh = q[b * SQ:(b + 1) * SQ, h * DH:(h + 1) * DH]
            k_bh = k_ref[b, :, h, :]
            v_bh = v_ref[b, :, h, :]
            s = lax.dot_general(
                q_bh, k_bh, (((1,), (1,)), ((), ())),
                preferred_element_type=jnp.float32,
            ) * SCALE
            m = jnp.max(s, axis=1)
            p = jnp.exp(s - m[:, None])
            l = jnp.sum(p, axis=1)
            o = jnp.dot(p, v_bh, preferred_element_type=jnp.float32)
            acc_o[idx] = o
            send_o[idx] = o.astype(jnp.bfloat16)
            acc_ml[0, idx] = m
            acc_ml[1, idx] = l

        HB = BH // 2

        def make_ml(r, partner):
            return pltpu.make_async_remote_copy(
                src_ref=acc_ml,
                dst_ref=recv_ml.at[r],
                send_sem=send_ml_sems.at[r],
                recv_sem=recv_ml_sems.at[r],
                device_id=(partner,),
                device_id_type=pl.DeviceIdType.MESH,
            )

        def make_o(r, c, partner):
            return pltpu.make_async_remote_copy(
                src_ref=send_o.at[pl.ds(c * HB, HB)],
                dst_ref=recv_o.at[r, pl.ds(c * HB, HB)],
                send_sem=send_o_sems.at[r, c],
                recv_sem=recv_o_sems.at[r, c],
                device_id=(partner,),
                device_id_type=pl.DeviceIdType.MESH,
            )

        partners = [my ^ (1 << r) for r in range(N_ROUNDS)]
        rdma_ml = {0: make_ml(0, partners[0])}
        rdma_o = {(0, 0): make_o(0, 0, partners[0]),
                  (0, 1): make_o(0, 1, partners[0])}

        for h in range(HQ):
            do_head(0, h)
        pl.semaphore_wait(barrier_sem, N_ROUNDS)
        rdma_o[0, 0].start()
        for h in range(HQ):
            do_head(1, h)
        rdma_ml[0].start()
        rdma_o[0, 1].start()

        for r in range(N_ROUNDS):
            rdma_ml[r].wait()
            m_a = acc_ml[0]
            l_a = acc_ml[1]
            m_b = recv_ml[r, 0]
            l_b = recv_ml[r, 1]
            m_n = jnp.maximum(m_a, m_b)
            alpha = jnp.exp(m_a - m_n)
            beta = jnp.exp(m_b - m_n)
            acc_ml[0] = m_n
            l_new = l_a * alpha + l_b * beta
            acc_ml[1] = l_new
            if r + 1 < N_ROUNDS:
                rdma_ml[r + 1] = make_ml(r + 1, partners[r + 1])
                rdma_ml[r + 1].start()

            for c in range(2):
                sl = pl.ds(c * HB, HB)
                csl = slice(c * HB, (c + 1) * HB)
                rdma_o[r, c].wait()
                merged = (
                    acc_o[sl] * alpha[csl, :, None]
                    + recv_o[r, sl].astype(jnp.float32)
                    * beta[csl, :, None]
                )
                if r + 1 < N_ROUNDS:
                    acc_o[sl] = merged
                    send_o[sl] = merged.astype(jnp.bfloat16)
                    rdma_o[r + 1, c] = make_o(r + 1, c, partners[r + 1])
                    rdma_o[r + 1, c].start()
                else:
                    o_norm = merged / l_new[csl, :, None]
                    for h in range(HQ):
                        attn_ref[c * SQ:(c + 1) * SQ,
                                 h * DH:(h + 1) * DH] = o_norm[h]
                    out_ref[c] = jnp.dot(
                        attn_ref[c * SQ:(c + 1) * SQ, :], wo_ref[:],
                        preferred_element_type=jnp.float32,
                    )

    return pl.pallas_call(
        body,
        out_shape=jax.ShapeDtypeStruct((B, SQ, D), jnp.float32),
        in_specs=[pl.BlockSpec(memory_space=pltpu.VMEM)] * 5,
        out_specs=pl.BlockSpec(memory_space=pltpu.VMEM),
        scratch_shapes=[
            pltpu.VMEM((BH, SQ, DH), jnp.float32),
            pltpu.VMEM((2, BH, SQ), jnp.float32),
            pltpu.VMEM((BH, SQ, DH), jnp.bfloat16),
            pltpu.VMEM((N_ROUNDS, BH, SQ, DH), jnp.bfloat16),
            pltpu.VMEM((N_ROUNDS, 2, BH, SQ), jnp.float32),
            pltpu.VMEM((B * SQ, HQ * DH), jnp.float32),
            pltpu.SemaphoreType.DMA((N_ROUNDS, 2)),
            pltpu.SemaphoreType.DMA((N_ROUNDS, 2)),
            pltpu.SemaphoreType.DMA((N_ROUNDS,)),
            pltpu.SemaphoreType.DMA((N_ROUNDS,)),
        ],
        compiler_params=pltpu.CompilerParams(collective_id=0),
    )(x, Wq, Wo, K_ext, V_ext)


# device time: 50621 ns/iter; 2.2615x vs baseline; 1.0119x over previous
import jax
import jax.numpy as jnp
from jax import lax
from jax.experimental import pallas as pl
from jax.experimental.pallas import tpu as pltpu

N_DEV = 32
N_ROUNDS = 5
B, SQ, D, HQ, DH = 2, 128, 512, 8, 64
SKV = 128
BH = B * HQ
SCALE = 0.125


def kernel(x, Wq, Wo, K_ext, V_ext):
    def body(
        x_ref, wq_ref, wo_ref, k_ref, v_ref,
        out_ref,
        acc_o, acc_ml, send_o, recv_o, recv_ml, attn_ref,
        send_o_sems, recv_o_sems, send_ml_sems, recv_ml_sems,
    ):
        my = lax.axis_index("i")

        barrier_sem = pltpu.get_barrier_semaphore()
        for r in range(N_ROUNDS):
            pl.semaphore_signal(
                barrier_sem, inc=1,
                device_id=(my ^ (1 << r),),
                device_id_type=pl.DeviceIdType.MESH,
            )

        xm = x_ref[:].reshape(B * SQ, D)

        def do_head(b, h, qmat):
            idx = b * HQ + h
            q_bh = qmat[:, h * DH:(h + 1) * DH]
            k_bh = k_ref[b, :, h, :]
            v_bh = v_ref[b, :, h, :]
            s = lax.dot_general(
                q_bh, k_bh, (((1,), (1,)), ((), ())),
                preferred_element_type=jnp.float32,
            ) * SCALE
            m = jnp.max(s, axis=1)
            p = jnp.exp(s - m[:, None])
            l = jnp.sum(p, axis=1)
            o = jnp.dot(p, v_bh, preferred_element_type=jnp.float32)
            acc_o[idx] = o
            send_o[idx] = o.astype(jnp.bfloat16)
            acc_ml[0, idx] = m
            acc_ml[1, idx] = l

        NC = 4
        HPC = BH // NC

        def make_ml(r, partner):
            return pltpu.make_async_remote_copy(
                src_ref=acc_ml,
                dst_ref=recv_ml.at[r],
                send_sem=send_ml_sems.at[r],
                recv_sem=recv_ml_sems.at[r],
                device_id=(partner,),
                device_id_type=pl.DeviceIdType.MESH,
            )

        def make_o(r, c, partner):
            return pltpu.make_async_remote_copy(
                src_ref=send_o.at[pl.ds(c * HPC, HPC)],
                dst_ref=recv_o.at[r, pl.ds(c * HPC, HPC)],
                send_sem=send_o_sems.at[r, c],
                recv_sem=recv_o_sems.at[r, c],
                device_id=(partner,),
                device_id_type=pl.DeviceIdType.MESH,
            )

        partners = [my ^ (1 << r) for r in range(N_ROUNDS)]
        rdma_ml = {0: make_ml(0, partners[0])}
        rdma_o = {(0, c): make_o(0, c, partners[0]) for c in range(NC)}

        q0 = jnp.dot(xm[0:SQ], wq_ref[:], preferred_element_type=jnp.float32)
        for h in range(HQ):
            do_head(0, h, q0)
        pl.semaphore_wait(barrier_sem, N_ROUNDS)
        rdma_o[0, 0].start()
        rdma_o[0, 1].start()
        q1 = jnp.dot(xm[SQ:], wq_ref[:], preferred_element_type=jnp.float32)
        for h in range(HQ):
            do_head(1, h, q1)
        rdma_ml[0].start()
        rdma_o[0, 2].start()
        rdma_o[0, 3].start()

        for r in range(N_ROUNDS):
            rdma_ml[r].wait()
            m_a = acc_ml[0]
            l_a = acc_ml[1]
            m_b = recv_ml[r, 0]
            l_b = recv_ml[r, 1]
            m_n = jnp.maximum(m_a, m_b)
            alpha = jnp.exp(m_a - m_n)
            beta = jnp.exp(m_b - m_n)
            acc_ml[0] = m_n
            l_new = l_a * alpha + l_b * beta
            acc_ml[1] = l_new
            if r + 1 < N_ROUNDS:
                rdma_ml[r + 1] = make_ml(r + 1, partners[r + 1])
                rdma_ml[r + 1].start()

            final_chunks = []
            for c in range(NC):
                sl = pl.ds(c * HPC, HPC)
                csl = slice(c * HPC, (c + 1) * HPC)
                rdma_o[r, c].wait()
                merged = (
                    acc_o[sl] * alpha[csl, :, None]
                    + recv_o[r, sl].astype(jnp.float32)
                    * beta[csl, :, None]
                )
                if r + 1 < N_ROUNDS:
                    acc_o[sl] = merged
                    send_o[sl] = merged.astype(jnp.bfloat16)
                    rdma_o[r + 1, c] = make_o(r + 1, c, partners[r + 1])
                    rdma_o[r + 1, c].start()
                else:
                    final_chunks.append(merged)
                    if c % 2 == 1:
                        b = c // 2
                        o_b = jnp.concatenate(final_chunks[-2:], axis=0)
                        o_norm = o_b / l_new[b * HQ:(b + 1) * HQ, :, None]
                        for h in range(HQ):
                            attn_ref[b * SQ:(b + 1) * SQ,
                                     h * DH:(h + 1) * DH] = o_norm[h]
                        out_ref[b] = jnp.dot(
                            attn_ref[b * SQ:(b + 1) * SQ, :], wo_ref[:],
                            preferred_element_type=jnp.float32,
                        )

    return pl.pallas_call(
        body,
        out_shape=jax.ShapeDtypeStruct((B, SQ, D), jnp.float32),
        in_specs=[pl.BlockSpec(memory_space=pltpu.VMEM)] * 5,
        out_specs=pl.BlockSpec(memory_space=pltpu.VMEM),
        scratch_shapes=[
            pltpu.VMEM((BH, SQ, DH), jnp.float32),
            pltpu.VMEM((2, BH, SQ), jnp.float32),
            pltpu.VMEM((BH, SQ, DH), jnp.bfloat16),
            pltpu.VMEM((N_ROUNDS, BH, SQ, DH), jnp.bfloat16),
            pltpu.VMEM((N_ROUNDS, 2, BH, SQ), jnp.float32),
            pltpu.VMEM((B * SQ, HQ * DH), jnp.float32),
            pltpu.SemaphoreType.DMA((N_ROUNDS, 4)),
            pltpu.SemaphoreType.DMA((N_ROUNDS, 4)),
            pltpu.SemaphoreType.DMA((N_ROUNDS,)),
            pltpu.SemaphoreType.DMA((N_ROUNDS,)),
        ],
        compiler_params=pltpu.CompilerParams(collective_id=0),
    )(x, Wq, Wo, K_ext, V_ext)


# device time: 50596 ns/iter; 2.2626x vs baseline; 1.0005x over previous
import jax
import jax.numpy as jnp
from jax import lax
from jax.experimental import pallas as pl
from jax.experimental.pallas import tpu as pltpu

N_DEV = 32
N_ROUNDS = 5
B, SQ, D, HQ, DH = 2, 128, 512, 8, 64
SKV = 128
BH = B * HQ
SCALE = 0.125


def kernel(x, Wq, Wo, K_ext, V_ext):
    def body(
        x_ref, wq_ref, wo_ref, k_ref, v_ref,
        out_ref,
        acc_o, acc_ml, recv_o, recv_ml, attn_ref,
        send_o_sems, recv_o_sems, send_ml_sems, recv_ml_sems,
    ):
        my = lax.axis_index("i")

        barrier_sem = pltpu.get_barrier_semaphore()
        for r in range(N_ROUNDS):
            pl.semaphore_signal(
                barrier_sem, inc=1,
                device_id=(my ^ (1 << r),),
                device_id_type=pl.DeviceIdType.MESH,
            )

        xm = x_ref[:].reshape(B * SQ, D)

        def do_head(b, h, qmat):
            idx = b * HQ + h
            q_bh = qmat[:, h * DH:(h + 1) * DH]
            k_bh = k_ref[b, :, h, :]
            v_bh = v_ref[b, :, h, :]
            s = lax.dot_general(
                q_bh, k_bh, (((1,), (1,)), ((), ())),
                preferred_element_type=jnp.float32,
            ) * SCALE
            m = jnp.max(s, axis=1)
            p = jnp.exp(s - m[:, None])
            l = jnp.sum(p, axis=1)
            o = jnp.dot(p, v_bh, preferred_element_type=jnp.float32)
            acc_o[idx] = o.astype(jnp.bfloat16)
            acc_ml[0, idx] = m
            acc_ml[1, idx] = l

        NC = 4
        HPC = BH // NC

        def make_ml(r, partner):
            return pltpu.make_async_remote_copy(
                src_ref=acc_ml,
                dst_ref=recv_ml.at[r],
                send_sem=send_ml_sems.at[r],
                recv_sem=recv_ml_sems.at[r],
                device_id=(partner,),
                device_id_type=pl.DeviceIdType.MESH,
            )

        def make_o(r, c, partner):
            return pltpu.make_async_remote_copy(
                src_ref=acc_o.at[pl.ds(c * HPC, HPC)],
                dst_ref=recv_o.at[r, pl.ds(c * HPC, HPC)],
                send_sem=send_o_sems.at[r, c],
                recv_sem=recv_o_sems.at[r, c],
                device_id=(partner,),
                device_id_type=pl.DeviceIdType.MESH,
            )

        partners = [my ^ (1 << r) for r in range(N_ROUNDS)]
        rdma_ml = {0: make_ml(0, partners[0])}
        rdma_o = {(0, c): make_o(0, c, partners[0]) for c in range(NC)}

        q0 = jnp.dot(xm[0:SQ], wq_ref[:], preferred_element_type=jnp.float32)
        for h in range(HQ):
            do_head(0, h, q0)
        pl.semaphore_wait(barrier_sem, N_ROUNDS)
        rdma_o[0, 0].start()
        rdma_o[0, 1].start()
        q1 = jnp.dot(xm[SQ:], wq_ref[:], preferred_element_type=jnp.float32)
        for h in range(HQ):
            do_head(1, h, q1)
        rdma_ml[0].start()
        rdma_o[0, 2].start()
        rdma_o[0, 3].start()

        for r in range(N_ROUNDS):
            rdma_ml[r].wait()
            m_a = acc_ml[0]
            l_a = acc_ml[1]
            m_b = recv_ml[r, 0]
            l_b = recv_ml[r, 1]
            m_n = jnp.maximum(m_a, m_b)
            alpha = jnp.exp(m_a - m_n)
            beta = jnp.exp(m_b - m_n)
            acc_ml[0] = m_n
            l_new = l_a * alpha + l_b * beta
            acc_ml[1] = l_new
            if r + 1 < N_ROUNDS:
                rdma_ml[r + 1] = make_ml(r + 1, partners[r + 1])
                rdma_ml[r + 1].start()

            final_chunks = []
            for c in range(NC):
                sl = pl.ds(c * HPC, HPC)
                csl = slice(c * HPC, (c + 1) * HPC)
                rdma_o[r, c].wait()
                merged = (
                    acc_o[sl].astype(jnp.float32) * alpha[csl, :, None]
                    + recv_o[r, sl].astype(jnp.float32)
                    * beta[csl, :, None]
                )
                if r + 1 < N_ROUNDS:
                    acc_o[sl] = merged.astype(jnp.bfloat16)
                    rdma_o[r + 1, c] = make_o(r + 1, c, partners[r + 1])
                    rdma_o[r + 1, c].start()
                else:
                    final_chunks.append(merged)
                    if c % 2 == 1:
                        b = c // 2
                        o_b = jnp.concatenate(final_chunks[-2:], axis=0)
                        o_norm = o_b / l_new[b * HQ:(b + 1) * HQ, :, None]
                        for h in range(HQ):
                            attn_ref[b * SQ:(b + 1) * SQ,
                                     h * DH:(h + 1) * DH] = o_norm[h]
                        out_ref[b] = jnp.dot(
                            attn_ref[b * SQ:(b + 1) * SQ, :], wo_ref[:],
                            preferred_element_type=jnp.float32,
                        )

    return pl.pallas_call(
        body,
        out_shape=jax.ShapeDtypeStruct((B, SQ, D), jnp.float32),
        in_specs=[pl.BlockSpec(memory_space=pltpu.VMEM)] * 5,
        out_specs=pl.BlockSpec(memory_space=pltpu.VMEM),
        scratch_shapes=[
            pltpu.VMEM((BH, SQ, DH), jnp.bfloat16),
            pltpu.VMEM((2, BH, SQ), jnp.float32),
            pltpu.VMEM((N_ROUNDS, BH, SQ, DH), jnp.bfloat16),
            pltpu.VMEM((N_ROUNDS, 2, BH, SQ), jnp.float32),
            pltpu.VMEM((B * SQ, HQ * DH), jnp.float32),
            pltpu.SemaphoreType.DMA((N_ROUNDS, 4)),
            pltpu.SemaphoreType.DMA((N_ROUNDS, 4)),
            pltpu.SemaphoreType.DMA((N_ROUNDS,)),
            pltpu.SemaphoreType.DMA((N_ROUNDS,)),
        ],
        compiler_params=pltpu.CompilerParams(collective_id=0),
    )(x, Wq, Wo, K_ext, V_ext)
